# baseline (speedup 1.0000x reference)
import jax
import jax.numpy as jnp
from jax.experimental import pallas as pl
from jax.experimental.pallas import tpu as pltpu

_B, _Q, _C = 4, 900, 151
_ALPHA = 0.25

_LOG1P_COEF = (
    1.6936626e-06, 9.9983257e-01, -4.9720332e-01, 3.1504127e-01,
    -1.8901955e-01, 8.1523180e-02, -1.7029611e-02,
)


def _log1p_poly(u):
    acc = jnp.full_like(u, _LOG1P_COEF[-1])
    for c in _LOG1P_COEF[-2::-1]:
        acc = acc * u + c
    return acc


def _focal_elem(x, t):
    e = jnp.exp(-jnp.abs(x))
    ce = jnp.maximum(x, 0.0) - x * t + _log1p_poly(e)
    r = 1.0 / (1.0 + e)
    prob = jnp.where(x >= 0.0, r, 1.0 - r)
    om = prob + t * (1.0 - 2.0 * prob)
    alpha_t = (1.0 - _ALPHA) - (1.0 - 2.0 * _ALPHA) * t
    return alpha_t * ce * om * om


_CHUNK = 16
_NFULL = _Q // _CHUNK
_TAIL = _Q - _NFULL * _CHUNK


def _tc_body(x_hbm, t_hbm, out_ref, x_v, t_v, sems):
    copies = []
    for b in range(_B):
        cx = pltpu.make_async_copy(x_hbm.at[b], x_v.at[b], sems.at[0, b])
        ct = pltpu.make_async_copy(t_hbm.at[b], t_v.at[b], sems.at[1, b])
        cx.start()
        ct.start()
        copies.append((cx, ct))

    total = jnp.zeros((), jnp.float32)
    for b in range(_B):
        copies[b][0].wait()
        copies[b][1].wait()

        def step(k, acc):
            r0 = k * _CHUNK
            return acc + _focal_elem(
                x_v[b, pl.ds(r0, _CHUNK), :], t_v[b, pl.ds(r0, _CHUNK), :]
            )

        acc = jax.lax.fori_loop(
            0, _NFULL, step, jnp.zeros((_CHUNK, _C), jnp.float32), unroll=2
        )
        tail = _focal_elem(
            x_v[b, pl.ds(_NFULL * _CHUNK, _TAIL), :],
            t_v[b, pl.ds(_NFULL * _CHUNK, _TAIL), :],
        )
        total += jnp.sum(acc) + jnp.sum(tail)
    out_ref[0] = total


def kernel(outputs, targets, num_targets):
    total = pl.pallas_call(
        _tc_body,
        in_specs=[
            pl.BlockSpec(memory_space=pl.ANY),
            pl.BlockSpec(memory_space=pl.ANY),
        ],
        out_specs=pl.BlockSpec(memory_space=pltpu.SMEM),
        out_shape=jax.ShapeDtypeStruct((1,), jnp.float32),
        scratch_shapes=[
            pltpu.VMEM((_B, _Q, _C), jnp.float32),
            pltpu.VMEM((_B, _Q, _C), jnp.float32),
            pltpu.SemaphoreType.DMA((2, _B)),
        ],
    )(outputs, targets)
    return total[0] * (float(_Q) / num_targets)
